# Initial kernel scaffold; baseline (speedup 1.0000x reference)
import jax
import jax.numpy as jnp
from jax.experimental import pallas as pl


def kernel(x, edge_index, edge_type, comp1, bases1, root1, bias1, comp2, bases2, root2, bias2):
    raise NotImplementedError("write your pallas kernel here")


# speedup vs baseline: 56.1070x; 56.1070x over previous
import functools

import jax
import jax.numpy as jnp
from jax import lax
from jax.experimental import pallas as pl
from jax.experimental.pallas import tpu as pltpu
from jax.experimental.pallas import tpu_sc as plsc

D = 128
_C = 80
_L = 16


def _weights_body(c_ref, b_ref, w_ref):
    w_ref[...] = jnp.dot(c_ref[...], b_ref[...],
                         preferred_element_type=jnp.float32)


def _weights(comp, bases_flat):
    R = comp.shape[0]
    return pl.pallas_call(
        _weights_body,
        out_shape=jax.ShapeDtypeStruct((R, bases_flat.shape[1]), jnp.float32),
    )(comp, bases_flat)


def _relmm_body(x_ref, w_ref, h_ref):
    h_ref[0] = jnp.dot(x_ref[...], w_ref[0],
                       preferred_element_type=jnp.float32)


def _relmm(x, W, bn):
    N = x.shape[0]
    R = W.shape[0]
    nt = N // bn
    return pl.pallas_call(
        _relmm_body,
        grid=(nt, R),
        in_specs=[
            pl.BlockSpec((bn, D), lambda i, r: (i, 0)),
            pl.BlockSpec((1, D, D), lambda i, r: (r, 0, 0)),
        ],
        out_specs=pl.BlockSpec((1, bn, D), lambda i, r: (r, i, 0)),
        out_shape=jax.ShapeDtypeStruct((R, N, D), jnp.float32),
    )(x, W)


def _combine_body(x_ref, r_ref, b_ref, p0_ref, p1_ref, o_ref):
    acc = jnp.dot(x_ref[...], r_ref[...], preferred_element_type=jnp.float32)
    acc = acc + b_ref[...] + p0_ref[...] + p1_ref[...]
    o_ref[...] = jnp.maximum(acc, 0.0)


def _combine(x, root, bias2d, p0, p1, bn):
    N = x.shape[0]
    nt = N // bn
    return pl.pallas_call(
        _combine_body,
        grid=(nt,),
        in_specs=[
            pl.BlockSpec((bn, D), lambda i: (i, 0)),
            pl.BlockSpec((D, D), lambda i: (0, 0)),
            pl.BlockSpec((1, D), lambda i: (0, 0)),
            pl.BlockSpec((bn, D), lambda i: (i, 0)),
            pl.BlockSpec((bn, D), lambda i: (i, 0)),
        ],
        out_specs=pl.BlockSpec((bn, D), lambda i: (i, 0)),
        out_shape=jax.ShapeDtypeStruct((N, D), jnp.float32),
    )(x, root, bias2d, p0, p1)


def _edge_prep_body(N, R, E, NRp,
                    src_hbm, dst_hbm, et_hbm, gidx_hbm, scale_hbm,
                    cnt_sh, d_v, t_v, s_v, cidx_v, gidx_v, ones_v, cv_v,
                    sc_v, zbuf):
    cid = lax.axis_index("c")
    sid = lax.axis_index("s")
    wid = sid * 2 + cid

    zn = zbuf.shape[0]

    def _zfill(k, _):
        zbuf[pl.ds(k * _L, _L)] = jnp.zeros((_L,), jnp.float32)
        return 0

    lax.fori_loop(0, zn // _L, _zfill, 0)
    per_tile = NRp // 16
    base_z = sid * per_tile

    def _zdma(k, _):
        pltpu.sync_copy(zbuf.at[pl.ds(0, zn)],
                        cnt_sh.at[pl.ds(base_z + k * zn, zn)])
        return 0

    lax.fori_loop(0, per_tile // zn, _zdma, 0)

    for k in range(_C // _L):
        ones_v[pl.ds(k * _L, _L)] = jnp.ones((_L,), jnp.float32)

    plsc.subcore_barrier()

    ec_sc = E // 16
    base_a = sid * ec_sc

    def _count_chunk(c, _):
        off = base_a + c * _C
        pltpu.sync_copy(dst_hbm.at[pl.ds(off, _C)], d_v)
        pltpu.sync_copy(et_hbm.at[pl.ds(off, _C)], t_v)
        for k in range(_C // _L):
            sl = pl.ds(k * _L, _L)
            cidx_v[sl] = d_v[sl] * R + t_v[sl]
        pltpu.sync_copy(ones_v, cnt_sh.at[cidx_v], add=True)
        return 0

    lax.fori_loop(0, ec_sc // _C, _count_chunk, 0)

    plsc.subcore_barrier()

    ec_w = E // 32
    base_b = wid * ec_w

    def _prep_chunk(c, _):
        off = base_b + c * _C
        pltpu.sync_copy(src_hbm.at[pl.ds(off, _C)], s_v)
        pltpu.sync_copy(dst_hbm.at[pl.ds(off, _C)], d_v)
        pltpu.sync_copy(et_hbm.at[pl.ds(off, _C)], t_v)
        for k in range(_C // _L):
            sl = pl.ds(k * _L, _L)
            cidx_v[sl] = d_v[sl] * R + t_v[sl]
            gidx_v[sl] = t_v[sl] * N + s_v[sl]
        pltpu.sync_copy(cnt_sh.at[cidx_v], cv_v)
        for k in range(_C // _L):
            sl = pl.ds(k * _L, _L)
            sc_v[sl] = 1.0 / cv_v[sl]
        pltpu.sync_copy(gidx_v, gidx_hbm.at[pl.ds(off, _C)])
        pltpu.sync_copy(sc_v, scale_hbm.at[pl.ds(off, _C)])
        return 0

    lax.fori_loop(0, ec_w // _C, _prep_chunk, 0)


def _edge_prep(src, dst, et, N, R):
    E = src.shape[0]
    NRp = 512000 if N * R == 500000 else ((N * R + 255) // 256) * 256
    mesh = plsc.VectorSubcoreMesh(core_axis_name="c", subcore_axis_name="s")
    kfn = pl.kernel(
        functools.partial(_edge_prep_body, N, R, E, NRp),
        out_type=(jax.ShapeDtypeStruct((E,), jnp.int32),
                  jax.ShapeDtypeStruct((E,), jnp.float32)),
        mesh=mesh,
        scratch_types=[
            pltpu.VMEM_SHARED((NRp,), jnp.float32),
            pltpu.VMEM((_C,), jnp.int32),
            pltpu.VMEM((_C,), jnp.int32),
            pltpu.VMEM((_C,), jnp.int32),
            pltpu.VMEM((_C,), jnp.int32),
            pltpu.VMEM((_C,), jnp.int32),
            pltpu.VMEM((_C,), jnp.float32),
            pltpu.VMEM((_C,), jnp.float32),
            pltpu.VMEM((_C,), jnp.float32),
            pltpu.VMEM((16000,), jnp.float32),
        ],
    )
    return kfn(src, dst, et)


def _aggregate_body(N, E,
                    h_hbm, gidx_hbm, dst_hbm, scale_hbm, p0_hbm, p1_hbm,
                    acc_sh, gv, dv, sv, rows, zrow, sem):
    cid = lax.axis_index("c")
    sid = lax.axis_index("s")
    wid = sid * 2 + cid

    zr = zrow.shape[0]
    rpt = (N // 16 // 8) * 8
    last = N - 15 * rpt
    base_r = sid * rpt

    def _zfill(i, _):
        for j in range(D // _L):
            zrow[i, pl.ds(j * _L, _L)] = jnp.zeros((_L,), jnp.float32)
        return 0

    lax.fori_loop(0, zr, _zfill, 0)

    for k in range(rpt // zr):
        pltpu.sync_copy(zrow.at[pl.ds(0, zr)],
                        acc_sh.at[pl.ds(base_r + k * zr, zr)])

    @pl.when(sid == 15)
    def _():
        for k in range(last // zr - rpt // zr):
            pltpu.sync_copy(
                zrow.at[pl.ds(0, zr)],
                acc_sh.at[pl.ds(base_r + rpt + k * zr, zr)])
        tail = last % zr
        if tail:
            pltpu.sync_copy(
                zrow.at[pl.ds(0, tail)],
                acc_sh.at[pl.ds(15 * rpt + last - tail, tail)])

    plsc.subcore_barrier()

    ec_w = E // 32
    base_e = wid * ec_w

    def _edge_chunk(c, _):
        off = base_e + c * _C
        pltpu.sync_copy(gidx_hbm.at[pl.ds(off, _C)], gv)
        pltpu.sync_copy(dst_hbm.at[pl.ds(off, _C)], dv)
        pltpu.sync_copy(scale_hbm.at[pl.ds(off, _C)], sv)
        pltpu.async_copy(h_hbm.at[gv], rows, sem).wait()

        def _scale_group(g, _):
            s16 = sv[pl.ds(g * _L, _L)]
            for k in range(_L):
                e = g * _L + k
                s = s16[k]
                for j in range(D // _L):
                    sl = pl.ds(j * _L, _L)
                    rows[e, sl] = rows[e, sl] * s
            return 0

        lax.fori_loop(0, _C // _L, _scale_group, 0)
        pltpu.sync_copy(rows, acc_sh.at[dv], add=True)
        return 0

    lax.fori_loop(0, ec_w // _C, _edge_chunk, 0)

    plsc.subcore_barrier()

    @pl.when((cid == 0) & (sid < 15))
    def _():
        pltpu.sync_copy(acc_sh.at[pl.ds(base_r, rpt)],
                        p0_hbm.at[pl.ds(base_r, rpt)])

    @pl.when((cid == 0) & (sid == 15))
    def _():
        pltpu.sync_copy(acc_sh.at[pl.ds(15 * rpt, last)],
                        p0_hbm.at[pl.ds(15 * rpt, last)])

    @pl.when((cid == 1) & (sid < 15))
    def _():
        pltpu.sync_copy(acc_sh.at[pl.ds(base_r, rpt)],
                        p1_hbm.at[pl.ds(base_r, rpt)])

    @pl.when((cid == 1) & (sid == 15))
    def _():
        pltpu.sync_copy(acc_sh.at[pl.ds(15 * rpt, last)],
                        p1_hbm.at[pl.ds(15 * rpt, last)])


def _aggregate(h_flat, gidx, dst, scale, N):
    E = gidx.shape[0]
    mesh = plsc.VectorSubcoreMesh(core_axis_name="c", subcore_axis_name="s")
    kfn = pl.kernel(
        functools.partial(_aggregate_body, N, E),
        out_type=(jax.ShapeDtypeStruct((N, D), jnp.float32),
                  jax.ShapeDtypeStruct((N, D), jnp.float32)),
        mesh=mesh,
        scratch_types=[
            pltpu.VMEM_SHARED((N, D), jnp.float32),
            pltpu.VMEM((_C,), jnp.int32),
            pltpu.VMEM((_C,), jnp.int32),
            pltpu.VMEM((_C,), jnp.float32),
            pltpu.VMEM((_C, D), jnp.float32),
            pltpu.VMEM((208, D), jnp.float32),
            pltpu.SemaphoreType.DMA,
        ],
    )
    return kfn(h_flat, gidx, dst, scale)


def kernel(x, edge_index, edge_type, comp1, bases1, root1, bias1,
           comp2, bases2, root2, bias2):
    N, d_in = x.shape
    R, NB = comp1.shape
    E = edge_type.shape[0]

    src = edge_index[0].astype(jnp.int32)
    dst = edge_index[1].astype(jnp.int32)
    et = edge_type.astype(jnp.int32)

    bn = 1000 if N % 1000 == 0 else N // 10

    W1 = _weights(comp1, bases1.reshape(NB, d_in * D)).reshape(R, d_in, D)
    W2 = _weights(comp2, bases2.reshape(NB, D * D)).reshape(R, D, D)

    gidx, scale = _edge_prep(src, dst, et, N, R)

    h1 = _relmm(x, W1, bn).reshape(R * N, D)
    p0, p1 = _aggregate(h1, gidx, dst, scale, N)
    h = _combine(x, root1, bias1.reshape(1, D), p0, p1, bn)

    h2 = _relmm(h, W2, bn).reshape(R * N, D)
    q0, q1 = _aggregate(h2, gidx, dst, scale, N)
    out = _combine(h, root2, bias2.reshape(1, D), q0, q1, bn)
    return out
